# Initial kernel scaffold; baseline (speedup 1.0000x reference)
import jax
import jax.numpy as jnp
from jax.experimental import pallas as pl


def kernel(sparse_codes, batch):
    raise NotImplementedError("write your pallas kernel here")


# speedup vs baseline: 5.6811x; 5.6811x over previous
import functools

import jax
import jax.numpy as jnp
from jax import lax
from jax.experimental import pallas as pl
from jax.experimental.pallas import tpu as pltpu
from jax.experimental.pallas import tpu_sc as plsc

N_ROWS = 100000
D = 128
NUM_GRAPHS = 64
REUSE_WEIGHT = 0.01

NW = 32
CPW = 25
CHUNK = 128
N_CHUNKS = NW * CPW
N_FULL = N_ROWS // CHUNK
ACC_ROWS = 72
CW = 16
LAST_OFF = N_ROWS - CHUNK


def _sc_partial_sums(codes, idx2d, zeros_s, zeros_c, ones_c):
    mesh = plsc.VectorSubcoreMesh(core_axis_name="c", subcore_axis_name="s")

    @functools.partial(
        pl.kernel,
        mesh=mesh,
        out_type=[
            jax.ShapeDtypeStruct((2, ACC_ROWS, D), jnp.float32),
            jax.ShapeDtypeStruct((2, ACC_ROWS, CW), jnp.float32),
        ],
        scratch_types=[
            pltpu.VMEM((1, CPW, CHUNK), jnp.int32),
            pltpu.VMEM((CHUNK, D), jnp.float32),
            pltpu.VMEM((CHUNK, CW), jnp.float32),
            pltpu.VMEM_SHARED((ACC_ROWS, D), jnp.float32),
            pltpu.VMEM_SHARED((ACC_ROWS, CW), jnp.float32),
        ],
    )
    def sc_kernel(codes_hbm, idx_hbm, zs_hbm, zc_hbm, ones_hbm,
                  out_s_hbm, out_c_hbm,
                  idx_v, stage_v, ones_v, acc_sh, cnt_sh):
        cid = lax.axis_index("c")
        sid = lax.axis_index("s")
        wid = cid * 16 + sid

        @pl.when(sid == 0)
        def _():
            pltpu.sync_copy(zs_hbm, acc_sh)
            pltpu.sync_copy(zc_hbm, cnt_sh)

        pltpu.sync_copy(ones_hbm, ones_v)
        pltpu.sync_copy(idx_hbm.at[pl.ds(wid, 1)], idx_v)
        plsc.subcore_barrier()

        def body(j, carry):
            src = jnp.minimum((wid * CPW + j) * CHUNK, LAST_OFF)
            pltpu.sync_copy(codes_hbm.at[pl.ds(src, CHUNK)], stage_v)
            pltpu.sync_copy(stage_v, acc_sh.at[idx_v.at[0, j]], add=True)
            pltpu.sync_copy(ones_v, cnt_sh.at[idx_v.at[0, j]], add=True)
            return carry

        lax.fori_loop(0, CPW, body, 0)
        plsc.subcore_barrier()

        @pl.when(sid == 0)
        def _():
            pltpu.sync_copy(acc_sh, out_s_hbm.at[cid])
            pltpu.sync_copy(cnt_sh, out_c_hbm.at[cid])

    return sc_kernel(codes, idx2d, zeros_s, zeros_c, ones_c)


def _tc_finalize(partial_s, partial_c):
    def body(s_ref, c_ref, o_ref):
        s = s_ref[0, :NUM_GRAPHS, :] + s_ref[1, :NUM_GRAPHS, :]
        cnt = c_ref[0, :NUM_GRAPHS, :] + c_ref[1, :NUM_GRAPHS, :]
        m = s / cnt[:, 0:1]
        mu = jnp.mean(m, axis=0, keepdims=True)
        var = jnp.sum((m - mu) ** 2, axis=0) / (NUM_GRAPHS - 1)
        o_ref[0, 0] = -REUSE_WEIGHT * jnp.mean(var)

    return pl.pallas_call(
        body,
        out_shape=jax.ShapeDtypeStruct((1, 1), jnp.float32),
        out_specs=pl.BlockSpec(memory_space=pltpu.SMEM),
    )(partial_s, partial_c)


def kernel(sparse_codes, batch):
    b32 = batch.astype(jnp.int32)
    idx_main = b32[: N_FULL * CHUNK].reshape(N_FULL, CHUNK)
    tail_valid = N_ROWS - N_FULL * CHUNK
    idx_tail = jnp.concatenate(
        [jnp.full((CHUNK - tail_valid,), NUM_GRAPHS, jnp.int32),
         b32[N_FULL * CHUNK:]]).reshape(1, CHUNK)
    idx_pad = jnp.full((N_CHUNKS - N_FULL - 1, CHUNK), NUM_GRAPHS, jnp.int32)
    idx2d = jnp.concatenate([idx_main, idx_tail, idx_pad],
                            axis=0).reshape(NW, CPW, CHUNK)

    zeros_s = jnp.zeros((ACC_ROWS, D), jnp.float32)
    zeros_c = jnp.zeros((ACC_ROWS, CW), jnp.float32)
    ones_c = jnp.ones((CHUNK, CW), jnp.float32)

    ps, pc = _sc_partial_sums(sparse_codes, idx2d, zeros_s, zeros_c, ones_c)
    return _tc_finalize(ps, pc)[0, 0]


# speedup vs baseline: 7.1052x; 1.2507x over previous
import functools

import jax
import jax.numpy as jnp
from jax import lax
from jax.experimental import pallas as pl
from jax.experimental.pallas import tpu as pltpu
from jax.experimental.pallas import tpu_sc as plsc

N_ROWS = 100000
D = 128
NUM_GRAPHS = 64
REUSE_WEIGHT = 0.01

NW = 32
CPW = 25
CHUNK = 128
N_CHUNKS = NW * CPW
N_FULL = N_ROWS // CHUNK
ACC_ROWS = 72
CW = 16
LAST_OFF = N_ROWS - CHUNK


def _sc_partial_sums(codes, idx2d, zeros_s, zeros_c, ones_c):
    mesh = plsc.VectorSubcoreMesh(core_axis_name="c", subcore_axis_name="s")

    @functools.partial(
        pl.kernel,
        mesh=mesh,
        out_type=[
            jax.ShapeDtypeStruct((2, ACC_ROWS, D), jnp.float32),
            jax.ShapeDtypeStruct((2, ACC_ROWS, CW), jnp.float32),
        ],
        scratch_types=[
            pltpu.VMEM((1, CPW, CHUNK), jnp.int32),
            pltpu.VMEM((2, CHUNK, D), jnp.float32),
            pltpu.VMEM((CHUNK, CW), jnp.float32),
            pltpu.VMEM_SHARED((ACC_ROWS, D), jnp.float32),
            pltpu.VMEM_SHARED((ACC_ROWS, CW), jnp.float32),
            pltpu.SemaphoreType.DMA,
            pltpu.SemaphoreType.DMA,
        ],
    )
    def sc_kernel(codes_hbm, idx_hbm, zs_hbm, zc_hbm, ones_hbm,
                  out_s_hbm, out_c_hbm,
                  idx_v, stage_v, ones_v, acc_sh, cnt_sh, sem0, sem1):
        cid = lax.axis_index("c")
        sid = lax.axis_index("s")
        wid = cid * 16 + sid
        sems = (sem0, sem1)

        @pl.when(sid == 0)
        def _():
            pltpu.sync_copy(zs_hbm, acc_sh)
            pltpu.sync_copy(zc_hbm, cnt_sh)

        pltpu.sync_copy(ones_hbm, ones_v)
        pltpu.sync_copy(idx_hbm.at[pl.ds(wid, 1)], idx_v)
        plsc.subcore_barrier()

        def start_in(c, b):
            src = jnp.minimum((wid * CPW + c) * CHUNK, LAST_OFF)
            pltpu.async_copy(codes_hbm.at[pl.ds(src, CHUNK)], stage_v.at[b],
                             sems[b])

        def wait_in(b):
            pltpu.make_async_copy(codes_hbm.at[pl.ds(0, CHUNK)],
                                  stage_v.at[b], sems[b]).wait()

        def scatter(c, b):
            pltpu.sync_copy(stage_v.at[b], acc_sh.at[idx_v.at[0, c]],
                            add=True)
            pltpu.sync_copy(ones_v, cnt_sh.at[idx_v.at[0, c]], add=True)

        start_in(0, 0)

        def body(i, carry):
            base = i * 2
            wait_in(0)
            start_in(base + 1, 1)
            scatter(base, 0)
            wait_in(1)
            start_in(base + 2, 0)
            scatter(base + 1, 1)
            return carry

        lax.fori_loop(0, (CPW - 1) // 2, body, 0)
        wait_in(0)
        scatter(CPW - 1, 0)
        plsc.subcore_barrier()

        @pl.when(sid == 0)
        def _():
            pltpu.sync_copy(acc_sh, out_s_hbm.at[cid])
            pltpu.sync_copy(cnt_sh, out_c_hbm.at[cid])

    return sc_kernel(codes, idx2d, zeros_s, zeros_c, ones_c)


def _tc_finalize(partial_s, partial_c):
    def body(s_ref, c_ref, o_ref):
        s = s_ref[0, :NUM_GRAPHS, :] + s_ref[1, :NUM_GRAPHS, :]
        cnt = c_ref[0, :NUM_GRAPHS, :] + c_ref[1, :NUM_GRAPHS, :]
        m = s / cnt[:, 0:1]
        mu = jnp.mean(m, axis=0, keepdims=True)
        var = jnp.sum((m - mu) ** 2, axis=0) / (NUM_GRAPHS - 1)
        o_ref[0, 0] = -REUSE_WEIGHT * jnp.mean(var)

    return pl.pallas_call(
        body,
        out_shape=jax.ShapeDtypeStruct((1, 1), jnp.float32),
        out_specs=pl.BlockSpec(memory_space=pltpu.SMEM),
    )(partial_s, partial_c)


def kernel(sparse_codes, batch):
    b32 = batch.astype(jnp.int32)
    idx_main = b32[: N_FULL * CHUNK].reshape(N_FULL, CHUNK)
    tail_valid = N_ROWS - N_FULL * CHUNK
    idx_tail = jnp.concatenate(
        [jnp.full((CHUNK - tail_valid,), NUM_GRAPHS, jnp.int32),
         b32[N_FULL * CHUNK:]]).reshape(1, CHUNK)
    idx_pad = jnp.full((N_CHUNKS - N_FULL - 1, CHUNK), NUM_GRAPHS, jnp.int32)
    idx2d = jnp.concatenate([idx_main, idx_tail, idx_pad],
                            axis=0).reshape(NW, CPW, CHUNK)

    zeros_s = jnp.zeros((ACC_ROWS, D), jnp.float32)
    zeros_c = jnp.zeros((ACC_ROWS, CW), jnp.float32)
    ones_c = jnp.ones((CHUNK, CW), jnp.float32)

    ps, pc = _sc_partial_sums(sparse_codes, idx2d, zeros_s, zeros_c, ones_c)
    return _tc_finalize(ps, pc)[0, 0]
